# baseline (speedup 1.0000x reference)
import functools

import jax
import jax.numpy as jnp
from jax import lax
from jax.experimental import pallas as pl
from jax.experimental.pallas import tpu as pltpu
from jax.experimental.pallas import tpu_sc as plsc

_ROW_BLOCK = 256
_WINDOW_EDGES = (0, 2816, 5632, 8192)
_GATHER_CHUNK = 128


def _argmin_body(x_ref, e_ref, xsq_ref, esq_ref, colsf_ref, idx_ref):
    sim = lax.dot_general(
        x_ref[...], e_ref[...], (((1,), (0,)), ((), ())),
        preferred_element_type=jnp.float32,
    )
    d = (xsq_ref[...] + esq_ref[...]) - 2.0 * sim
    n = d.shape[1]
    cols = colsf_ref[...]
    acc_v = None
    for lo, hi in zip(_WINDOW_EDGES[:-1], _WINDOW_EDGES[1:]):
        dw = d[:, lo:hi]
        wv = jnp.min(dw, axis=1, keepdims=True)
        wi = jnp.min(
            jnp.where(dw == wv, cols[:, lo:hi], jnp.float32(n)),
            axis=1, keepdims=True,
        )
        if acc_v is None:
            acc_v, acc_i = wv, wi
        else:
            win = wv < acc_v
            acc_i = jnp.where(win, wi, acc_i)
            acc_v = jnp.where(win, wv, acc_v)
        acc_v = acc_v.astype(jnp.bfloat16).astype(jnp.float32)
    idx_ref[...] = acc_i.astype(jnp.int32)


def _tc_argmin(xf, emb, xsq, esq):
    b, d = xf.shape
    n = emb.shape[1]
    rb = _ROW_BLOCK
    return pl.pallas_call(
        _argmin_body,
        grid=(b // rb,),
        in_specs=[
            pl.BlockSpec((rb, d), lambda r: (r, 0)),
            pl.BlockSpec((d, n), lambda r: (0, 0)),
            pl.BlockSpec((rb, 1), lambda r: (r, 0)),
            pl.BlockSpec((1, n), lambda r: (0, 0)),
            pl.BlockSpec((1, n), lambda r: (0, 0)),
        ],
        out_specs=pl.BlockSpec((rb, 1), lambda r: (r, 0)),
        out_shape=jax.ShapeDtypeStruct((b, 1), jnp.int32),
    )(xf, emb, xsq, esq, lax.iota(jnp.float32, n).reshape(1, n))


def _sc_gather(table, idx):
    n, d = table.shape
    b = idx.shape[0]
    info = plsc.get_sparse_core_info()
    nw = info.num_cores * info.num_subcores
    ch = _GATHER_CHUNK
    per_w = b // nw
    n_ch = per_w // ch
    mesh = plsc.VectorSubcoreMesh(core_axis_name="c", subcore_axis_name="s")

    @functools.partial(
        pl.kernel,
        mesh=mesh,
        out_type=jax.ShapeDtypeStruct((b, d), jnp.float32),
        scratch_types=[
            pltpu.VMEM((ch,), jnp.int32),
            pltpu.VMEM((ch,), jnp.int32),
            pltpu.VMEM((ch, d), jnp.float32),
            pltpu.VMEM((ch, d), jnp.float32),
            pltpu.SemaphoreType.DMA,
            pltpu.SemaphoreType.DMA,
        ],
    )
    def k(table_hbm, idx_hbm, out_hbm, idx_v0, idx_v1, rows_v0, rows_v1, sem0, sem1):
        wid = lax.axis_index("s") * info.num_cores + lax.axis_index("c")
        idx_v = (idx_v0, idx_v1)
        rows_v = (rows_v0, rows_v1)
        sem = (sem0, sem1)
        copies = [None, None]
        for c in range(n_ch):
            s = c % 2
            base = wid * per_w + c * ch
            pltpu.sync_copy(idx_hbm.at[pl.ds(base, ch)], idx_v[s])
            copies[s] = pltpu.async_copy(table_hbm.at[idx_v[s]], rows_v[s], sem[s])
            if c > 0:
                copies[1 - s].wait()
                prev = wid * per_w + (c - 1) * ch
                pltpu.sync_copy(rows_v[1 - s], out_hbm.at[pl.ds(prev, ch)])
        last = n_ch - 1
        copies[last % 2].wait()
        pltpu.sync_copy(rows_v[last % 2],
                        out_hbm.at[pl.ds(wid * per_w + last * ch, ch)])

    return k(table, idx)


def kernel(x, embeddings):
    d = embeddings.shape[0]
    b = x.size // d
    xf = x.reshape(b, d)
    xsq = jnp.sum(xf**2, axis=1, keepdims=True)
    esq = jnp.sum(embeddings**2, axis=0, keepdims=True)
    idx = _tc_argmin(xf, embeddings, xsq, esq)
    quantized = _sc_gather(embeddings.T, idx.reshape(b))
    return quantized.reshape(x.shape)


# speedup vs baseline: 1.1597x; 1.1597x over previous
import functools

import jax
import jax.numpy as jnp
from jax import lax
from jax.experimental import pallas as pl
from jax.experimental.pallas import tpu as pltpu
from jax.experimental.pallas import tpu_sc as plsc

_ROW_BLOCK = 1024
_WINDOW_EDGES = (0, 2816, 5632, 8192)
_GATHER_CHUNK = 128


def _argmin_body(x_ref, e_ref, xsq_ref, esq_ref, colsf_ref, idx_ref):
    sim = lax.dot_general(
        x_ref[...], e_ref[...], (((1,), (0,)), ((), ())),
        preferred_element_type=jnp.float32,
    )
    d = (xsq_ref[...] + esq_ref[...]) - 2.0 * sim
    n = d.shape[1]
    cols = colsf_ref[...]
    acc_v = None
    for lo, hi in zip(_WINDOW_EDGES[:-1], _WINDOW_EDGES[1:]):
        dw = d[:, lo:hi]
        wv = jnp.min(dw, axis=1, keepdims=True)
        wi = jnp.min(
            jnp.where(dw == wv, cols[:, lo:hi], jnp.float32(n)),
            axis=1, keepdims=True,
        )
        if acc_v is None:
            acc_v, acc_i = wv, wi
        else:
            win = wv < acc_v
            acc_i = jnp.where(win, wi, acc_i)
            acc_v = jnp.where(win, wv, acc_v)
        acc_v = acc_v.astype(jnp.bfloat16).astype(jnp.float32)
    idx_ref[...] = acc_i.astype(jnp.int32)


def _tc_argmin(xf, emb, xsq, esq):
    b, d = xf.shape
    n = emb.shape[1]
    rb = _ROW_BLOCK
    return pl.pallas_call(
        _argmin_body,
        grid=(b // rb,),
        in_specs=[
            pl.BlockSpec((rb, d), lambda r: (r, 0)),
            pl.BlockSpec((d, n), lambda r: (0, 0)),
            pl.BlockSpec((rb, 1), lambda r: (r, 0)),
            pl.BlockSpec((1, n), lambda r: (0, 0)),
            pl.BlockSpec((1, n), lambda r: (0, 0)),
        ],
        out_specs=pl.BlockSpec((rb, 1), lambda r: (r, 0)),
        out_shape=jax.ShapeDtypeStruct((b, 1), jnp.int32),
    )(xf, emb, xsq, esq, lax.iota(jnp.float32, n).reshape(1, n))


def _sc_gather(table, idx):
    n, d = table.shape
    b = idx.shape[0]
    info = plsc.get_sparse_core_info()
    nw = info.num_cores * info.num_subcores
    ch = _GATHER_CHUNK
    per_w = b // nw
    n_ch = per_w // ch
    mesh = plsc.VectorSubcoreMesh(core_axis_name="c", subcore_axis_name="s")

    @functools.partial(
        pl.kernel,
        mesh=mesh,
        out_type=jax.ShapeDtypeStruct((b, d), jnp.float32),
        scratch_types=[
            pltpu.VMEM((ch,), jnp.int32),
            pltpu.VMEM((ch,), jnp.int32),
            pltpu.VMEM((ch, d), jnp.float32),
            pltpu.VMEM((ch, d), jnp.float32),
            pltpu.SemaphoreType.DMA,
            pltpu.SemaphoreType.DMA,
        ],
    )
    def k(table_hbm, idx_hbm, out_hbm, idx_v0, idx_v1, rows_v0, rows_v1, sem0, sem1):
        wid = lax.axis_index("s") * info.num_cores + lax.axis_index("c")
        idx_v = (idx_v0, idx_v1)
        rows_v = (rows_v0, rows_v1)
        sem = (sem0, sem1)
        copies = [None, None]
        for c in range(n_ch):
            s = c % 2
            base = wid * per_w + c * ch
            pltpu.sync_copy(idx_hbm.at[pl.ds(base, ch)], idx_v[s])
            copies[s] = pltpu.async_copy(table_hbm.at[idx_v[s]], rows_v[s], sem[s])
            if c > 0:
                copies[1 - s].wait()
                prev = wid * per_w + (c - 1) * ch
                pltpu.sync_copy(rows_v[1 - s], out_hbm.at[pl.ds(prev, ch)])
        last = n_ch - 1
        copies[last % 2].wait()
        pltpu.sync_copy(rows_v[last % 2],
                        out_hbm.at[pl.ds(wid * per_w + last * ch, ch)])

    return k(table, idx)


def kernel(x, embeddings):
    d = embeddings.shape[0]
    b = x.size // d
    xf = x.reshape(b, d)
    xsq = jnp.sum(xf**2, axis=1, keepdims=True)
    esq = jnp.sum(embeddings**2, axis=0, keepdims=True)
    idx = _tc_argmin(xf, embeddings, xsq, esq)
    quantized = _sc_gather(embeddings.T, idx.reshape(b))
    return quantized.reshape(x.shape)
